# baseline (speedup 1.0000x reference)
import jax
import jax.numpy as jnp
from jax.experimental import pallas as pl
from jax.experimental.pallas import tpu as pltpu

_ROWS, _COLS = 16384, 1024
_CB = 4096
_NC = _ROWS // _CB
_K = 3
_L = 1


def _dma_pipeline(x_hbm, y_hbm, o_hbm, buf, yv, in_sems, out_sems, ysem):
    cy = pltpu.make_async_copy(y_hbm.at[pl.ds(0, 8)], yv, ysem)
    cy.start()

    ins = [None] * _NC
    outs = [None] * _NC

    def start_in(c):
        ins[c] = pltpu.make_async_copy(
            x_hbm.at[pl.ds(c * _CB, _CB)],
            buf.at[pl.ds((c % _K) * _CB, _CB)],
            in_sems.at[c],
        )
        ins[c].start()

    def start_out(c):
        ins[c].wait()
        if c == 0:
            cy.wait()
            r = jax.lax.broadcasted_iota(jnp.int32, (8, _COLS), 0)
            cc = jax.lax.broadcasted_iota(jnp.int32, (8, _COLS), 1)
            buf[0:8, :] = jnp.where((r == 0) & (cc == 3), yv[0, 2], buf[0:8, :])
        outs[c] = pltpu.make_async_copy(
            buf.at[pl.ds((c % _K) * _CB, _CB)],
            o_hbm.at[pl.ds(c * _CB, _CB)],
            out_sems.at[c],
        )
        outs[c].start()

    for c in range(_NC):
        if c >= _K:
            outs[c - _K].wait()
        start_in(c)
        if c >= _L:
            start_out(c - _L)
    for c in range(_NC - _L, _NC):
        start_out(c)
    for c in range(max(_NC - _K, 0), _NC):
        outs[c].wait()


def kernel(x, y):
    return pl.pallas_call(
        _dma_pipeline,
        in_specs=[
            pl.BlockSpec(memory_space=pl.ANY),
            pl.BlockSpec(memory_space=pl.ANY),
        ],
        out_specs=pl.BlockSpec(memory_space=pl.ANY),
        out_shape=jax.ShapeDtypeStruct((_ROWS, _COLS), x.dtype),
        scratch_shapes=[
            pltpu.VMEM((_K * _CB, _COLS), jnp.float32),
            pltpu.VMEM((8, _COLS), jnp.float32),
            pltpu.SemaphoreType.DMA((_NC,)),
            pltpu.SemaphoreType.DMA((_NC,)),
            pltpu.SemaphoreType.DMA,
        ],
    )(x, y)


# speedup vs baseline: 1.0126x; 1.0126x over previous
import jax
import jax.numpy as jnp
from jax.experimental import pallas as pl
from jax.experimental.pallas import tpu as pltpu

_ROWS, _COLS = 16384, 1024
_CB = 2048
_NC = _ROWS // _CB
_K = 7
_L = 2


def _dma_pipeline(x_hbm, y_hbm, o_hbm, buf, yv, in_sems, out_sems, ysem):
    cy = pltpu.make_async_copy(y_hbm.at[pl.ds(0, 8)], yv, ysem)
    cy.start()

    ins = [None] * _NC
    outs = [None] * _NC

    def start_in(c):
        ins[c] = pltpu.make_async_copy(
            x_hbm.at[pl.ds(c * _CB, _CB)],
            buf.at[pl.ds((c % _K) * _CB, _CB)],
            in_sems.at[c],
        )
        ins[c].start()

    def start_out(c):
        ins[c].wait()
        if c == 0:
            cy.wait()
            r = jax.lax.broadcasted_iota(jnp.int32, (8, _COLS), 0)
            cc = jax.lax.broadcasted_iota(jnp.int32, (8, _COLS), 1)
            buf[0:8, :] = jnp.where((r == 0) & (cc == 3), yv[0, 2], buf[0:8, :])
        outs[c] = pltpu.make_async_copy(
            buf.at[pl.ds((c % _K) * _CB, _CB)],
            o_hbm.at[pl.ds(c * _CB, _CB)],
            out_sems.at[c],
        )
        outs[c].start()

    for c in range(_NC):
        if c >= _K:
            outs[c - _K].wait()
        start_in(c)
        if c >= _L:
            start_out(c - _L)
    for c in range(_NC - _L, _NC):
        start_out(c)
    for c in range(max(_NC - _K, 0), _NC):
        outs[c].wait()


def kernel(x, y):
    return pl.pallas_call(
        _dma_pipeline,
        in_specs=[
            pl.BlockSpec(memory_space=pl.ANY),
            pl.BlockSpec(memory_space=pl.ANY),
        ],
        out_specs=pl.BlockSpec(memory_space=pl.ANY),
        out_shape=jax.ShapeDtypeStruct((_ROWS, _COLS), x.dtype),
        scratch_shapes=[
            pltpu.VMEM((_K * _CB, _COLS), jnp.float32),
            pltpu.VMEM((8, _COLS), jnp.float32),
            pltpu.SemaphoreType.DMA((_NC,)),
            pltpu.SemaphoreType.DMA((_NC,)),
            pltpu.SemaphoreType.DMA,
        ],
    )(x, y)


# speedup vs baseline: 1.0276x; 1.0147x over previous
import jax
import jax.numpy as jnp
from jax.experimental import pallas as pl
from jax.experimental.pallas import tpu as pltpu

_ROWS, _COLS = 16384, 1024
_CB = 2048
_NC = _ROWS // _CB
_K = 7
_L = 4


def _dma_pipeline(x_hbm, y_hbm, o_hbm, buf, yv, in_sems, out_sems, ysem):
    cy = pltpu.make_async_copy(y_hbm.at[pl.ds(0, 8)], yv, ysem)
    cy.start()

    ins = [None] * _NC
    outs = [None] * _NC

    def start_in(c):
        ins[c] = pltpu.make_async_copy(
            x_hbm.at[pl.ds(c * _CB, _CB)],
            buf.at[pl.ds((c % _K) * _CB, _CB)],
            in_sems.at[c],
        )
        ins[c].start()

    def start_out(c):
        ins[c].wait()
        if c == 0:
            cy.wait()
            r = jax.lax.broadcasted_iota(jnp.int32, (8, _COLS), 0)
            cc = jax.lax.broadcasted_iota(jnp.int32, (8, _COLS), 1)
            buf[0:8, :] = jnp.where((r == 0) & (cc == 3), yv[0, 2], buf[0:8, :])
        outs[c] = pltpu.make_async_copy(
            buf.at[pl.ds((c % _K) * _CB, _CB)],
            o_hbm.at[pl.ds(c * _CB, _CB)],
            out_sems.at[c],
        )
        outs[c].start()

    for c in range(_NC):
        if c >= _K:
            outs[c - _K].wait()
        start_in(c)
        if c >= _L:
            start_out(c - _L)
    for c in range(_NC - _L, _NC):
        start_out(c)
    for c in range(max(_NC - _K, 0), _NC):
        outs[c].wait()


def kernel(x, y):
    return pl.pallas_call(
        _dma_pipeline,
        in_specs=[
            pl.BlockSpec(memory_space=pl.ANY),
            pl.BlockSpec(memory_space=pl.ANY),
        ],
        out_specs=pl.BlockSpec(memory_space=pl.ANY),
        out_shape=jax.ShapeDtypeStruct((_ROWS, _COLS), x.dtype),
        scratch_shapes=[
            pltpu.VMEM((_K * _CB, _COLS), jnp.float32),
            pltpu.VMEM((8, _COLS), jnp.float32),
            pltpu.SemaphoreType.DMA((_NC,)),
            pltpu.SemaphoreType.DMA((_NC,)),
            pltpu.SemaphoreType.DMA,
        ],
    )(x, y)


# speedup vs baseline: 1.0407x; 1.0128x over previous
import jax
import jax.numpy as jnp
from jax.experimental import pallas as pl
from jax.experimental.pallas import tpu as pltpu

_ROWS, _COLS = 16384, 1024
_CB = 2048
_NC = _ROWS // _CB
_K = 7
_L = 5


def _dma_pipeline(x_hbm, y_hbm, o_hbm, buf, yv, in_sems, out_sems, ysem):
    cy = pltpu.make_async_copy(y_hbm.at[pl.ds(0, 8)], yv, ysem)
    cy.start()

    ins = [None] * _NC
    outs = [None] * _NC

    def start_in(c):
        ins[c] = pltpu.make_async_copy(
            x_hbm.at[pl.ds(c * _CB, _CB)],
            buf.at[pl.ds((c % _K) * _CB, _CB)],
            in_sems.at[c],
        )
        ins[c].start()

    def start_out(c):
        ins[c].wait()
        if c == 0:
            cy.wait()
            r = jax.lax.broadcasted_iota(jnp.int32, (8, _COLS), 0)
            cc = jax.lax.broadcasted_iota(jnp.int32, (8, _COLS), 1)
            buf[0:8, :] = jnp.where((r == 0) & (cc == 3), yv[0, 2], buf[0:8, :])
        outs[c] = pltpu.make_async_copy(
            buf.at[pl.ds((c % _K) * _CB, _CB)],
            o_hbm.at[pl.ds(c * _CB, _CB)],
            out_sems.at[c],
        )
        outs[c].start()

    for c in range(_NC):
        if c >= _K:
            outs[c - _K].wait()
        start_in(c)
        if c >= _L:
            start_out(c - _L)
    for c in range(_NC - _L, _NC):
        start_out(c)
    for c in range(max(_NC - _K, 0), _NC):
        outs[c].wait()


def kernel(x, y):
    return pl.pallas_call(
        _dma_pipeline,
        in_specs=[
            pl.BlockSpec(memory_space=pl.ANY),
            pl.BlockSpec(memory_space=pl.ANY),
        ],
        out_specs=pl.BlockSpec(memory_space=pl.ANY),
        out_shape=jax.ShapeDtypeStruct((_ROWS, _COLS), x.dtype),
        scratch_shapes=[
            pltpu.VMEM((_K * _CB, _COLS), jnp.float32),
            pltpu.VMEM((8, _COLS), jnp.float32),
            pltpu.SemaphoreType.DMA((_NC,)),
            pltpu.SemaphoreType.DMA((_NC,)),
            pltpu.SemaphoreType.DMA,
        ],
    )(x, y)
